# Initial kernel scaffold; baseline (speedup 1.0000x reference)
import jax
import jax.numpy as jnp
from jax.experimental import pallas as pl


def kernel(pred_grad, gt_grad):
    raise NotImplementedError("write your pallas kernel here")


# speedup vs baseline: 7.1532x; 7.1532x over previous
import jax
import jax.numpy as jnp
import numpy as np
from jax import lax
from jax.experimental import pallas as pl
from jax.experimental.pallas import tpu as pltpu
from jax.experimental.pallas import tpu_sc as plsc

BINS = 64
MARGIN = 0.4
N = 512 * 512
NSUB = 16
CH = N // NSUB
NV = CH // 16
K_RANK = int(np.float32(0.95) * np.float32(N - 1))
Q_FRAC = float(np.float32(0.95) * np.float32(N - 1)) - K_RANK

_f32 = jnp.float32
_i32 = jnp.int32


def _body(pred_hbm, gt_hbm, out_hbm,
          gtbuf, pdbuf, cnt, loc256, hist2, loc64, locv, rbmin, rbhist,
          s_cnt, s_min, s_hist):
    c = lax.axis_index("c")
    s = lax.axis_index("s")
    lane = lax.iota(_i32, 16)
    ones_i = jnp.full((16,), 1, _i32)
    zeros_i = jnp.full((16,), 0, _i32)
    zeros_f = jnp.full((16,), 0.0, _f32)

    def lane_reduce(src, nreg, nlane, dst):
        def red(cc, _):
            acc = zeros_i if src.dtype == _i32 else zeros_f
            for l in range(nlane):
                acc = acc + src[pl.ds(l * nreg * 16 + cc * 16, 16)]
            dst[pl.ds(cc * 16, 16)] = acc
            return 0
        lax.fori_loop(0, nreg, red, 0)

    loss_total = _f32(0.0)
    for img in range(2):
        row = (2 * c + img) * NSUB + s
        pltpu.sync_copy(gt_hbm.at[row], gtbuf)
        pltpu.sync_copy(pred_hbm.at[row], pdbuf)

        prefix = _i32(0)
        count_before = _i32(0)
        c_le = _i32(0)
        for p in range(4):
            sh = 24 - 8 * p

            def zero_cnt(kk, _):
                cnt[pl.ds(kk * 16, 16)] = zeros_i
                return 0
            lax.fori_loop(0, 256, zero_cnt, 0)

            def scan(kk, _):
                v = gtbuf[pl.ds(kk * 16, 16)]
                bits = lax.bitcast_convert_type(v, _i32)
                b = lax.shift_right_logical(bits, sh) & 255
                idx = lane * 256 + b
                if p == 0:
                    plsc.addupdate_scatter(cnt, [idx], ones_i)
                else:
                    m = lax.shift_right_logical(bits, sh + 8) == prefix
                    plsc.addupdate_scatter(cnt, [idx], ones_i, mask=m)
                return 0
            lax.fori_loop(0, NV, scan, 0)

            lane_reduce(cnt, 16, 16, loc256)
            pltpu.sync_copy(loc256, s_cnt.at[pl.ds(s * 256, 256)])
            plsc.subcore_barrier()
            pltpu.sync_copy(s_cnt, cnt)

            r_loc = K_RANK - count_before

            def select(cc, carry):
                done, bin_, running, cb, cle = carry
                h = zeros_i
                for w in range(NSUB):
                    h = h + cnt[pl.ds(w * 256 + cc * 16, 16)]
                s_inc = plsc.cumsum(h)
                tot = jnp.sum(h)
                crossed = (running + s_inc) >= (r_loc + 1)
                anyc = jnp.sum(jnp.where(crossed, 1, 0)) > 0
                nfalse = jnp.sum(jnp.where(crossed, 0, 1))
                e_inc = jnp.sum(jnp.where(lane == nfalse, s_inc, 0))
                e_exc = e_inc - jnp.sum(jnp.where(lane == nfalse, h, 0))
                hit = jnp.logical_and(done == 0, anyc)
                bin_ = jnp.where(hit, cc * 16 + nfalse, bin_)
                cle = jnp.where(hit, running + e_inc, cle)
                cb = jnp.where(hit, running + e_exc, cb)
                done = jnp.where(hit, _i32(1), done)
                return done, bin_, running + tot, cb, cle

            _, bin_, _, cb, cle = lax.fori_loop(
                0, 16, select, (_i32(0), _i32(0), _i32(0), _i32(0), _i32(0)))
            prefix = (prefix << 8) | bin_
            count_before = cb
            c_le = cle
            plsc.subcore_barrier()

        vk = lax.bitcast_convert_type(jnp.full((16,), prefix, _i32), _f32)

        inf_v = jnp.full((16,), jnp.inf, _f32)

        def minscan(kk, mm):
            v = gtbuf[pl.ds(kk * 16, 16)]
            return jnp.minimum(mm, jnp.where(v > vk, v, inf_v))
        mm = lax.fori_loop(0, NV, minscan, inf_v)
        locv[pl.ds(0, 16)] = mm
        pltpu.sync_copy(locv, s_min.at[pl.ds(s * 16, 16)])
        plsc.subcore_barrier()
        pltpu.sync_copy(s_min, rbmin)
        gmv = inf_v
        for w in range(NSUB):
            gmv = jnp.minimum(gmv, rbmin[pl.ds(w * 16, 16)])
        gm = jnp.min(gmv)
        have_next = jnp.full((16,), c_le, _i32) >= (K_RANK + 2)
        vnext = jnp.where(have_next, vk, jnp.full((16,), gm, _f32))
        max_val = vk + _f32(Q_FRAC) * (vnext - vk)
        scale = _f32(BINS) / max_val

        for ti, buf in ((0, pdbuf), (1, gtbuf)):
            def zero_h(kk, _):
                hist2[pl.ds(kk * 16, 16)] = zeros_f
                return 0
            lax.fori_loop(0, BINS, zero_h, 0)

            def hscan(kk, _):
                v = buf[pl.ds(kk * 16, 16)]
                t = jnp.minimum(v * scale, _f32(65.0))
                j = t.astype(_i32)
                fr = t - j.astype(_f32)
                idx = lane * BINS + j
                plsc.addupdate_scatter(hist2, [idx], _f32(1.0) - fr,
                                       mask=j <= BINS - 1)
                plsc.addupdate_scatter(hist2, [idx + 1], fr,
                                       mask=j <= BINS - 2)
                return 0
            lax.fori_loop(0, NV, hscan, 0)

            lane_reduce(hist2, 4, 16, loc64)
            pltpu.sync_copy(loc64, s_hist.at[pl.ds((s * 2 + ti) * BINS, BINS)])
        plsc.subcore_barrier()

        @pl.when(s == 0)
        def _():
            pltpu.sync_copy(s_hist, rbhist)
            hp = []
            hg = []
            for cc in range(BINS // 16):
                accp = zeros_f
                accg = zeros_f
                for w in range(NSUB):
                    accp = accp + rbhist[pl.ds((w * 2 + 0) * BINS + cc * 16, 16)]
                    accg = accg + rbhist[pl.ds((w * 2 + 1) * BINS + cc * 16, 16)]
                hp.append(accp)
                hg.append(accg)
            psum = _f32(0.0)
            gsum = _f32(0.0)
            for cc in range(BINS // 16):
                psum = psum + jnp.sum(hp[cc])
                gsum = gsum + jnp.sum(hg[cc])
            ones_f = jnp.full((16,), 1.0, _f32)
            pinv = ones_f / jnp.full((16,), psum, _f32)
            ginv = ones_f / jnp.full((16,), gsum, _f32)
            li = _f32(0.0)
            for cc in range(BINS // 16):
                jbin = (lane + cc * 16).astype(_f32)
                wgt = jnp.exp(_f32(MARGIN) * jbin * _f32(1.0 / BINS))
                diff = jnp.abs(hp[cc] * pinv * wgt - hg[cc] * ginv * wgt)
                li = li + jnp.sum(diff)
            locv[pl.ds(0, 16)] = jnp.full((16,), li * _f32(1.0 / BINS), _f32)
        plsc.subcore_barrier()
        if img == 0:
            loss_total = locv[pl.ds(0, 16)]
        else:
            loss_total = loss_total + locv[pl.ds(0, 16)]

    @pl.when(s == 0)
    def _():
        locv[pl.ds(0, 16)] = loss_total
        pltpu.sync_copy(locv, out_hbm.at[c])


def kernel(pred_grad, gt_grad):
    pred2 = pred_grad.reshape(4 * NSUB, CH)
    gt2 = gt_grad.reshape(4 * NSUB, CH)
    mesh = plsc.VectorSubcoreMesh(core_axis_name="c", subcore_axis_name="s")
    k = pl.kernel(
        _body,
        out_type=jax.ShapeDtypeStruct((2, 16), _f32),
        mesh=mesh,
        compiler_params=pltpu.CompilerParams(needs_layout_passes=False),
        scratch_types=[
            pltpu.VMEM((CH,), _f32),
            pltpu.VMEM((CH,), _f32),
            pltpu.VMEM((4096,), _i32),
            pltpu.VMEM((256,), _i32),
            pltpu.VMEM((16 * BINS,), _f32),
            pltpu.VMEM((BINS,), _f32),
            pltpu.VMEM((16,), _f32),
            pltpu.VMEM((256,), _f32),
            pltpu.VMEM((2 * NSUB * BINS,), _f32),
            pltpu.VMEM_SHARED((NSUB * 256,), _i32),
            pltpu.VMEM_SHARED((NSUB * 16,), _f32),
            pltpu.VMEM_SHARED((2 * NSUB * BINS,), _f32),
        ],
    )
    out = k(pred2, gt2)
    return (out[0, 0] + out[1, 0]) * _f32(0.25)


# speedup vs baseline: 16.9512x; 2.3698x over previous
import jax
import jax.numpy as jnp
import numpy as np
from jax import lax
from jax.experimental import pallas as pl
from jax.experimental.pallas import tpu as pltpu
from jax.experimental.pallas import tpu_sc as plsc

BINS = 64
MARGIN = 0.4
N = 512 * 512
NSUB = 16
CH = N // NSUB
K_RANK = int(np.float32(0.95) * np.float32(N - 1))
Q_FRAC = float(np.float32(0.95) * np.float32(N - 1)) - K_RANK

_f32 = jnp.float32
_i32 = jnp.int32


def _body(pred_hbm, gt_hbm, out_hbm,
          gtbuf, pdbuf, cnt, loc256, hist2, loc64, locv, rbmin,
          rbhist, s_cnt, s_min, s_hist):
    c = lax.axis_index("c")
    s = lax.axis_index("s")
    lane = lax.iota(_i32, 16)
    ones_i = jnp.full((16,), 1, _i32)
    zeros_i = jnp.full((16,), 0, _i32)
    zeros_f = jnp.full((16,), 0.0, _f32)

    loss_total = zeros_f
    for img in range(2):
        row = (2 * c + img) * NSUB + s
        pltpu.sync_copy(gt_hbm.at[row], gtbuf)
        pltpu.sync_copy(pred_hbm.at[row], pdbuf)

        prefix = _i32(0)
        count_before = _i32(0)
        c_le = _i32(0)
        for p in range(4):
            sh = 24 - 8 * p

            @plsc.parallel_loop(0, 4096, step=16, unroll=4, carry=_i32(0))
            def _(kk, cy):
                cnt[pl.ds(kk, 16)] = zeros_i
                return cy

            @plsc.parallel_loop(0, CH, step=16, unroll=8, carry=_i32(0))
            def _(kk, cy):
                v = gtbuf[pl.ds(kk, 16)]
                bits = lax.bitcast_convert_type(v, _i32)
                b = lax.shift_right_logical(bits, sh) & 255
                idx = lane * 256 + b
                if p == 0:
                    plsc.addupdate_scatter(cnt, [idx], ones_i)
                else:
                    m = lax.shift_right_logical(bits, sh + 8) == prefix
                    plsc.addupdate_scatter(cnt, [idx], ones_i, mask=m)
                return cy

            @plsc.parallel_loop(0, 256, step=16, unroll=2, carry=_i32(0))
            def _(cc, cy):
                acc = zeros_i
                for l in range(16):
                    acc = acc + cnt[pl.ds(l * 256 + cc, 16)]
                loc256[pl.ds(cc, 16)] = acc
                return cy

            pltpu.sync_copy(loc256, s_cnt.at[pl.ds(s * 256, 256)])
            plsc.subcore_barrier()
            pltpu.sync_copy(s_cnt, cnt)

            r_loc = K_RANK - count_before

            def select(cc, carry):
                done, bin_, running, cb, cle = carry
                h = zeros_i
                for w in range(NSUB):
                    h = h + cnt[pl.ds(w * 256 + cc * 16, 16)]
                s_inc = plsc.cumsum(h)
                tot = jnp.sum(h)
                crossed = (running + s_inc) >= (r_loc + 1)
                anyc = jnp.sum(jnp.where(crossed, 1, 0)) > 0
                nfalse = jnp.sum(jnp.where(crossed, 0, 1))
                e_inc = jnp.sum(jnp.where(lane == nfalse, s_inc, 0))
                e_exc = e_inc - jnp.sum(jnp.where(lane == nfalse, h, 0))
                hit = jnp.logical_and(done == 0, anyc)
                bin_ = jnp.where(hit, cc * 16 + nfalse, bin_)
                cle = jnp.where(hit, running + e_inc, cle)
                cb = jnp.where(hit, running + e_exc, cb)
                done = jnp.where(hit, _i32(1), done)
                return done, bin_, running + tot, cb, cle

            _, bin_, _, cb, cle = lax.fori_loop(
                0, 16, select, (_i32(0), _i32(0), _i32(0), _i32(0), _i32(0)))
            prefix = (prefix << 8) | bin_
            count_before = cb
            c_le = cle
            plsc.subcore_barrier()

        vk = lax.bitcast_convert_type(jnp.full((16,), prefix, _i32), _f32)

        inf_v = jnp.full((16,), jnp.inf, _f32)

        @plsc.parallel_loop(0, CH, step=16, unroll=8, carry=inf_v)
        def mm(kk, m):
            v = gtbuf[pl.ds(kk, 16)]
            return jnp.minimum(m, jnp.where(v > vk, v, inf_v))
        locv[pl.ds(0, 16)] = mm
        pltpu.sync_copy(locv, s_min.at[pl.ds(s * 16, 16)])
        plsc.subcore_barrier()
        pltpu.sync_copy(s_min, rbmin)
        gmv = inf_v
        for w in range(NSUB):
            gmv = jnp.minimum(gmv, rbmin[pl.ds(w * 16, 16)])
        gm = jnp.min(gmv)
        have_next = jnp.full((16,), c_le, _i32) >= (K_RANK + 2)
        vnext = jnp.where(have_next, vk, jnp.full((16,), gm, _f32))
        max_val = vk + _f32(Q_FRAC) * (vnext - vk)
        scale = _f32(BINS) / max_val

        for ti, buf in ((0, pdbuf), (1, gtbuf)):
            @plsc.parallel_loop(0, 16 * BINS, step=16, unroll=4, carry=_i32(0))
            def _(kk, cy):
                hist2[pl.ds(kk, 16)] = zeros_f
                return cy

            @plsc.parallel_loop(0, CH, step=16, unroll=8, carry=_i32(0))
            def _(kk, cy):
                v = buf[pl.ds(kk, 16)]
                t = jnp.minimum(v * scale, _f32(65.0))
                j = t.astype(_i32)
                fr = t - j.astype(_f32)
                idx = lane * BINS + j
                plsc.addupdate_scatter(hist2, [idx], _f32(1.0) - fr,
                                       mask=j <= BINS - 1)
                plsc.addupdate_scatter(hist2, [idx + 1], fr,
                                       mask=j <= BINS - 2)
                return cy

            @plsc.parallel_loop(0, BINS, step=16, unroll=1, carry=_i32(0))
            def _(cc, cy):
                acc = zeros_f
                for l in range(16):
                    acc = acc + hist2[pl.ds(l * BINS + cc, 16)]
                loc64[pl.ds(cc, 16)] = acc
                return cy

            pltpu.sync_copy(loc64, s_hist.at[pl.ds((s * 2 + ti) * BINS, BINS)])
        plsc.subcore_barrier()

        @pl.when(s == 0)
        def _():
            pltpu.sync_copy(s_hist, rbhist)
            hp = []
            hg = []
            for cc in range(BINS // 16):
                accp = zeros_f
                accg = zeros_f
                for w in range(NSUB):
                    accp = accp + rbhist[pl.ds((w * 2 + 0) * BINS + cc * 16, 16)]
                    accg = accg + rbhist[pl.ds((w * 2 + 1) * BINS + cc * 16, 16)]
                hp.append(accp)
                hg.append(accg)
            psum = _f32(0.0)
            gsum = _f32(0.0)
            for cc in range(BINS // 16):
                psum = psum + jnp.sum(hp[cc])
                gsum = gsum + jnp.sum(hg[cc])
            ones_f = jnp.full((16,), 1.0, _f32)
            pinv = ones_f / jnp.full((16,), psum, _f32)
            ginv = ones_f / jnp.full((16,), gsum, _f32)
            li = _f32(0.0)
            for cc in range(BINS // 16):
                jbin = (lane + cc * 16).astype(_f32)
                wgt = jnp.exp(_f32(MARGIN) * jbin * _f32(1.0 / BINS))
                diff = jnp.abs(hp[cc] * pinv * wgt - hg[cc] * ginv * wgt)
                li = li + jnp.sum(diff)
            locv[pl.ds(0, 16)] = jnp.full((16,), li * _f32(1.0 / BINS), _f32)
        plsc.subcore_barrier()
        loss_total = loss_total + locv[pl.ds(0, 16)]

    @pl.when(s == 0)
    def _():
        locv[pl.ds(0, 16)] = loss_total
        pltpu.sync_copy(locv, out_hbm.at[c])


def kernel(pred_grad, gt_grad):
    pred2 = pred_grad.reshape(4 * NSUB, CH)
    gt2 = gt_grad.reshape(4 * NSUB, CH)
    mesh = plsc.VectorSubcoreMesh(core_axis_name="c", subcore_axis_name="s")
    k = pl.kernel(
        _body,
        out_type=jax.ShapeDtypeStruct((2, 16), _f32),
        mesh=mesh,
        compiler_params=pltpu.CompilerParams(needs_layout_passes=False),
        scratch_types=[
            pltpu.VMEM((CH,), _f32),
            pltpu.VMEM((CH,), _f32),
            pltpu.VMEM((4096,), _i32),
            pltpu.VMEM((256,), _i32),
            pltpu.VMEM((16 * BINS,), _f32),
            pltpu.VMEM((BINS,), _f32),
            pltpu.VMEM((16,), _f32),
            pltpu.VMEM((256,), _f32),
            pltpu.VMEM((2 * NSUB * BINS,), _f32),
            pltpu.VMEM_SHARED((NSUB * 256,), _i32),
            pltpu.VMEM_SHARED((NSUB * 16,), _f32),
            pltpu.VMEM_SHARED((2 * NSUB * BINS,), _f32),
        ],
    )
    out = k(pred2, gt2)
    return (out[0, 0] + out[1, 0]) * _f32(0.25)
